# baseline (speedup 1.0000x reference)
import functools

import jax
import jax.numpy as jnp
from jax import lax
from jax.experimental import pallas as pl
from jax.experimental.pallas import tpu as pltpu
from jax.experimental.pallas import tpu_sc as plsc


@functools.lru_cache(maxsize=None)
def _build(samples: int, time: int, dim: int):
    info = plsc.get_sparse_core_info()
    nw = info.num_cores * info.num_subcores
    sub_blocks = dim // 8
    lane_chunks = nw // sub_blocks
    assert dim % 8 == 0 and time % (128 * lane_chunks) == 0
    cols = time // lane_chunks

    mesh = plsc.VectorSubcoreMesh(core_axis_name="c", subcore_axis_name="s")

    @functools.partial(
        pl.kernel,
        out_type=jax.ShapeDtypeStruct((samples, dim, time), jnp.float32),
        mesh=mesh,
        scratch_types=[
            pltpu.SemaphoreType.DMA,
        ],
        compiler_params=pltpu.CompilerParams(use_tc_tiling_on_sc=True),
    )
    def tile_kernel(table_hbm, out_hbm, sem):
        wid = lax.axis_index("s") * info.num_cores + lax.axis_index("c")
        row = (wid // lane_chunks) * 8
        col = (wid % lane_chunks) * cols
        src = table_hbm.at[pl.ds(row, 8), pl.ds(col, cols)]
        copies = [
            pltpu.async_copy(
                src, out_hbm.at[s, pl.ds(row, 8), pl.ds(col, cols)], sem
            )
            for s in range(samples)
        ]
        for c in copies:
            c.wait()

    return tile_kernel


def kernel(x, table):
    table_t = jnp.swapaxes(table, 0, 1)
    out_t = _build(x.shape[0], table.shape[0], table.shape[1])(table_t)
    return jnp.swapaxes(out_t, 1, 2)


# speedup vs baseline: 4.1680x; 4.1680x over previous
import functools

import jax
import jax.numpy as jnp
from jax import lax
from jax.experimental import pallas as pl
from jax.experimental.pallas import tpu as pltpu
from jax.experimental.pallas import tpu_sc as plsc


@functools.lru_cache(maxsize=None)
def _build(samples: int, time: int, dim: int):
    info = plsc.get_sparse_core_info()
    nc = info.num_cores
    rows = dim // nc

    mesh = plsc.ScalarSubcoreMesh(axis_name="c", num_cores=nc)

    @functools.partial(
        pl.kernel,
        out_type=jax.ShapeDtypeStruct((samples, dim, time), jnp.float32),
        mesh=mesh,
        scratch_types=[
            pltpu.VMEM_SHARED((rows, time), jnp.float32),
            pltpu.SemaphoreType.DMA,
        ],
        compiler_params=pltpu.CompilerParams(use_tc_tiling_on_sc=True),
    )
    def tile_kernel(table_hbm, out_hbm, spbuf, sem):
        row = lax.axis_index("c") * rows
        pltpu.sync_copy(table_hbm.at[pl.ds(row, rows), :], spbuf)
        copies = [
            pltpu.async_copy(spbuf, out_hbm.at[s, pl.ds(row, rows), :], sem)
            for s in range(samples)
        ]
        for c in copies:
            c.wait()

    return tile_kernel


def kernel(x, table):
    table_t = jnp.swapaxes(table, 0, 1)
    out_t = _build(x.shape[0], table.shape[0], table.shape[1])(table_t)
    return jnp.swapaxes(out_t, 1, 2)


# speedup vs baseline: 4.1725x; 1.0011x over previous
import functools

import jax
import jax.numpy as jnp
from jax import lax
from jax.experimental import pallas as pl
from jax.experimental.pallas import tpu as pltpu
from jax.experimental.pallas import tpu_sc as plsc


@functools.lru_cache(maxsize=None)
def _build(samples: int, time: int, dim: int):
    info = plsc.get_sparse_core_info()
    nc = info.num_cores
    nchunks = 2
    rows = dim // nc
    crows = rows // nchunks

    mesh = plsc.ScalarSubcoreMesh(axis_name="c", num_cores=nc)

    @functools.partial(
        pl.kernel,
        out_type=jax.ShapeDtypeStruct((samples, dim, time), jnp.float32),
        mesh=mesh,
        scratch_types=[
            pltpu.VMEM_SHARED((rows, time), jnp.float32),
            pltpu.SemaphoreType.DMA,
            pltpu.SemaphoreType.DMA,
        ],
        compiler_params=pltpu.CompilerParams(use_tc_tiling_on_sc=True),
    )
    def tile_kernel(table_hbm, out_hbm, spbuf, in_sem, out_sem):
        base = lax.axis_index("c") * rows
        loads = [
            pltpu.async_copy(
                table_hbm.at[pl.ds(base + k * crows, crows), :],
                spbuf.at[pl.ds(k * crows, crows), :],
                in_sem,
            )
            for k in range(nchunks)
        ]
        stores = []
        for k in range(nchunks):
            loads[k].wait()
            stores += [
                pltpu.async_copy(
                    spbuf.at[pl.ds(k * crows, crows), :],
                    out_hbm.at[s, pl.ds(base + k * crows, crows), :],
                    out_sem,
                )
                for s in range(samples)
            ]
        for c in stores:
            c.wait()

    return tile_kernel


def kernel(x, table):
    table_t = jnp.swapaxes(table, 0, 1)
    out_t = _build(x.shape[0], table.shape[0], table.shape[1])(table_t)
    return jnp.swapaxes(out_t, 1, 2)


# speedup vs baseline: 4.1830x; 1.0025x over previous
import functools

import jax
import jax.numpy as jnp
from jax import lax
from jax.experimental import pallas as pl
from jax.experimental.pallas import tpu as pltpu
from jax.experimental.pallas import tpu_sc as plsc


@functools.lru_cache(maxsize=None)
def _build(samples: int, time: int, dim: int):
    info = plsc.get_sparse_core_info()
    nc = info.num_cores
    nchunks = 2
    rows = dim // nc
    crows = rows // nchunks

    mesh = plsc.ScalarSubcoreMesh(axis_name="c", num_cores=nc)

    @functools.partial(
        pl.kernel,
        out_type=jax.ShapeDtypeStruct((samples, dim, time), jnp.float32),
        mesh=mesh,
        scratch_types=[
            pltpu.VMEM_SHARED((rows, time), jnp.float32),
            pltpu.SemaphoreType.DMA,
            pltpu.SemaphoreType.DMA,
        ],
        compiler_params=pltpu.CompilerParams(
            use_tc_tiling_on_sc=True, skip_device_barrier=True
        ),
    )
    def tile_kernel(table_hbm, out_hbm, spbuf, in_sem, out_sem):
        base = lax.axis_index("c") * rows
        loads = [
            pltpu.async_copy(
                table_hbm.at[pl.ds(base + k * crows, crows), :],
                spbuf.at[pl.ds(k * crows, crows), :],
                in_sem,
            )
            for k in range(nchunks)
        ]
        stores = []
        for k in range(nchunks):
            loads[k].wait()
            stores += [
                pltpu.async_copy(
                    spbuf.at[pl.ds(k * crows, crows), :],
                    out_hbm.at[s, pl.ds(base + k * crows, crows), :],
                    out_sem,
                )
                for s in range(samples)
            ]
        for c in stores:
            c.wait()

    return tile_kernel


def kernel(x, table):
    table_t = jnp.swapaxes(table, 0, 1)
    out_t = _build(x.shape[0], table.shape[0], table.shape[1])(table_t)
    return jnp.swapaxes(out_t, 1, 2)


# speedup vs baseline: 4.1847x; 1.0004x over previous
import functools

import jax
import jax.numpy as jnp
from jax import lax
from jax.experimental import pallas as pl
from jax.experimental.pallas import tpu as pltpu
from jax.experimental.pallas import tpu_sc as plsc


@functools.lru_cache(maxsize=None)
def _build(samples: int, time: int, dim: int):
    info = plsc.get_sparse_core_info()
    nc = 1
    nchunks = 2
    rows = dim // nc
    crows = rows // nchunks

    mesh = plsc.ScalarSubcoreMesh(axis_name="c", num_cores=nc)

    @functools.partial(
        pl.kernel,
        out_type=jax.ShapeDtypeStruct((samples, dim, time), jnp.float32),
        mesh=mesh,
        scratch_types=[
            pltpu.VMEM_SHARED((rows, time), jnp.float32),
            pltpu.SemaphoreType.DMA,
            pltpu.SemaphoreType.DMA,
        ],
        compiler_params=pltpu.CompilerParams(
            use_tc_tiling_on_sc=True, skip_device_barrier=True
        ),
    )
    def tile_kernel(table_hbm, out_hbm, spbuf, in_sem, out_sem):
        base = lax.axis_index("c") * rows
        loads = [
            pltpu.async_copy(
                table_hbm.at[pl.ds(base + k * crows, crows), :],
                spbuf.at[pl.ds(k * crows, crows), :],
                in_sem,
            )
            for k in range(nchunks)
        ]
        stores = []
        for k in range(nchunks):
            loads[k].wait()
            stores += [
                pltpu.async_copy(
                    spbuf.at[pl.ds(k * crows, crows), :],
                    out_hbm.at[s, pl.ds(base + k * crows, crows), :],
                    out_sem,
                )
                for s in range(samples)
            ]
        for c in stores:
            c.wait()

    return tile_kernel


def kernel(x, table):
    table_t = jnp.swapaxes(table, 0, 1)
    out_t = _build(x.shape[0], table.shape[0], table.shape[1])(table_t)
    return jnp.swapaxes(out_t, 1, 2)


# speedup vs baseline: 4.1949x; 1.0024x over previous
import functools

import jax
import jax.numpy as jnp
from jax import lax
from jax.experimental import pallas as pl
from jax.experimental.pallas import tpu as pltpu
from jax.experimental.pallas import tpu_sc as plsc


@functools.lru_cache(maxsize=None)
def _build(samples: int, time: int, dim: int):
    info = plsc.get_sparse_core_info()
    nc = 1
    nchunks = 4
    rows = dim // nc
    crows = rows // nchunks

    mesh = plsc.ScalarSubcoreMesh(axis_name="c", num_cores=nc)

    @functools.partial(
        pl.kernel,
        out_type=jax.ShapeDtypeStruct((samples, dim, time), jnp.float32),
        mesh=mesh,
        scratch_types=[
            pltpu.VMEM_SHARED((rows, time), jnp.float32),
            pltpu.SemaphoreType.DMA,
            pltpu.SemaphoreType.DMA,
        ],
        compiler_params=pltpu.CompilerParams(use_tc_tiling_on_sc=True),
    )
    def tile_kernel(table_hbm, out_hbm, spbuf, in_sem, out_sem):
        base = lax.axis_index("c") * rows
        loads = [
            pltpu.async_copy(
                table_hbm.at[pl.ds(base + k * crows, crows), :],
                spbuf.at[pl.ds(k * crows, crows), :],
                in_sem,
            )
            for k in range(nchunks)
        ]
        stores = []
        for k in range(nchunks):
            loads[k].wait()
            stores += [
                pltpu.async_copy(
                    spbuf.at[pl.ds(k * crows, crows), :],
                    out_hbm.at[s, pl.ds(base + k * crows, crows), :],
                    out_sem,
                )
                for s in range(samples)
            ]
        for c in stores:
            c.wait()

    return tile_kernel


def kernel(x, table):
    table_t = jnp.swapaxes(table, 0, 1)
    out_t = _build(x.shape[0], table.shape[0], table.shape[1])(table_t)
    return jnp.swapaxes(out_t, 1, 2)
